# baseline (speedup 1.0000x reference)
import functools

import jax
import jax.numpy as jnp
from jax import lax
from jax.experimental import pallas as pl
from jax.experimental.pallas import tpu as pltpu
from jax.experimental.pallas import tpu_sc as plsc

_NUM_ENT = 100000
_NUM_REL = 100000
_D = 64
_B = 16384
_L = 16
_NC = 2
_NS = 16
_NW = _NC * _NS
_BPW = _B // _NW
_C = 128
_NCHUNK = _BPW // _C
_G = _D // _L

_mesh = plsc.VectorSubcoreMesh(core_axis_name="c", subcore_axis_name="s")


@functools.partial(
    pl.kernel,
    mesh=_mesh,
    compiler_params=pltpu.CompilerParams(
        needs_layout_passes=False, use_tc_tiling_on_sc=False),
    out_type=[
        jax.ShapeDtypeStruct((_B,), jnp.float32),
        jax.ShapeDtypeStruct((_NW, _L), jnp.float32),
    ],
    scratch_types=[
        pltpu.VMEM((_C,), jnp.int32),
        pltpu.VMEM((_C,), jnp.int32),
        pltpu.VMEM((_C,), jnp.int32),
        pltpu.VMEM((_C, _D), jnp.float32),
        pltpu.VMEM((_C, _D), jnp.float32),
        pltpu.VMEM((_C, _D), jnp.float32),
        pltpu.VMEM((_C, _D), jnp.float32),
        pltpu.VMEM((_C, _D), jnp.float32),
        pltpu.VMEM((_C, _D), jnp.float32),
        pltpu.VMEM((_C,), jnp.float32),
        pltpu.VMEM((_L,), jnp.float32),
        pltpu.SemaphoreType.DMA,
    ],
)
def _sc_score(heads, rels, tails, ent_re, ent_im, rel_re, rel_im,
              score_out, sq_out,
              idx_h, idx_r, idx_t, bhre, bhim, btre, btim, brre, brim,
              score_buf, sq_buf, sem):
    wid = lax.axis_index("s") * _NC + lax.axis_index("c")
    base = wid * _BPW
    lane = lax.iota(jnp.int32, _L)

    def chunk_body(ci, sqacc):
        off = base + ci * _C
        pltpu.sync_copy(heads.at[pl.ds(off, _C)], idx_h)
        pltpu.sync_copy(rels.at[pl.ds(off, _C)], idx_r)
        pltpu.sync_copy(tails.at[pl.ds(off, _C)], idx_t)
        cps = [
            pltpu.async_copy(ent_re.at[idx_h], bhre, sem),
            pltpu.async_copy(ent_im.at[idx_h], bhim, sem),
            pltpu.async_copy(ent_re.at[idx_t], btre, sem),
            pltpu.async_copy(ent_im.at[idx_t], btim, sem),
            pltpu.async_copy(rel_re.at[idx_r], brre, sem),
            pltpu.async_copy(rel_im.at[idx_r], brim, sem),
        ]
        for cp in cps:
            cp.wait()

        def blk_body(blk, sqacc):
            i0 = blk * _L
            rows = i0 + lane
            acc = jnp.zeros((_L,), jnp.float32)
            for d in range(_D):
                col = jnp.full((_L,), d, jnp.int32)
                hre = plsc.load_gather(bhre, [rows, col])
                him = plsc.load_gather(bhim, [rows, col])
                tre = plsc.load_gather(btre, [rows, col])
                tim = plsc.load_gather(btim, [rows, col])
                rre = plsc.load_gather(brre, [rows, col])
                rim = plsc.load_gather(brim, [rows, col])
                acc = acc + rre * (hre * tre + him * tim)
                acc = acc + rim * (hre * tim - him * tre)
                sqacc = sqacc + hre * hre + him * him
                sqacc = sqacc + tre * tre + tim * tim
                sqacc = sqacc + rre * rre + rim * rim
            score_buf[pl.ds(i0, _L)] = -acc
            return sqacc

        sqacc = lax.fori_loop(0, _C // _L, blk_body, sqacc)
        pltpu.sync_copy(score_buf, score_out.at[pl.ds(off, _C)])
        return sqacc

    sqacc = lax.fori_loop(0, _NCHUNK, chunk_body, jnp.zeros((_L,), jnp.float32))
    sq_buf[...] = sqacc
    pltpu.sync_copy(sq_buf, sq_out.at[wid])


def _tc_finish_body(score_ref, labels_ref, sq_ref, out_ref):
    x = score_ref[...] * labels_ref[...]
    sp = jnp.maximum(x, 0.0) + jnp.log(1.0 + jnp.exp(-jnp.abs(x)))
    regul = jnp.sum(sq_ref[...]) * (0.01 / (_B * _D))
    total = jnp.sum(sp) * (1.0 / _B) + regul
    out_ref[...] = jnp.broadcast_to(total, (1, 1))


def _tc_finish(score2d, labels2d, sq):
    return pl.pallas_call(
        _tc_finish_body,
        out_shape=jax.ShapeDtypeStruct((1, 1), jnp.float32),
    )(score2d, labels2d, sq)


def kernel(batch, labels, ent_re, ent_im, rel_re, rel_im):
    heads = batch[:, 0]
    rels = batch[:, 1]
    tails = batch[:, 2]
    score, sq = _sc_score(heads, rels, tails, ent_re, ent_im, rel_re, rel_im)
    loss = _tc_finish(score.reshape(128, 128), labels.reshape(128, 128), sq)
    return loss[0, 0]


# speedup vs baseline: 1.2660x; 1.2660x over previous
import functools

import jax
import jax.numpy as jnp
from jax import lax
from jax.experimental import pallas as pl
from jax.experimental.pallas import tpu as pltpu
from jax.experimental.pallas import tpu_sc as plsc

_NUM_ENT = 100000
_NUM_REL = 100000
_D = 64
_B = 16384
_L = 16
_NC = 2
_NS = 16
_NW = _NC * _NS
_BPW = _B // _NW
_C = 128
_NCHUNK = _BPW // _C

_mesh = plsc.VectorSubcoreMesh(core_axis_name="c", subcore_axis_name="s")


@functools.partial(
    pl.kernel,
    mesh=_mesh,
    compiler_params=pltpu.CompilerParams(needs_layout_passes=False),
    out_type=[
        jax.ShapeDtypeStruct((_B,), jnp.float32),
        jax.ShapeDtypeStruct((_NW, _L), jnp.float32),
    ],
    scratch_types=[
        pltpu.SMEM((3 * _BPW,), jnp.int32),
        pltpu.VMEM((_BPW,), jnp.int32),
        pltpu.VMEM((_BPW,), jnp.int32),
        pltpu.VMEM((_BPW,), jnp.int32),
        pltpu.VMEM_SHARED((_NS, 3 * _BPW), jnp.int32),
        pltpu.VMEM((_C, _D), jnp.float32),
        pltpu.VMEM((_C, _D), jnp.float32),
        pltpu.VMEM((_C, _D), jnp.float32),
        pltpu.VMEM((_C, _D), jnp.float32),
        pltpu.VMEM((_C, _D), jnp.float32),
        pltpu.VMEM((_C, _D), jnp.float32),
        pltpu.VMEM((_C,), jnp.float32),
        pltpu.VMEM((_L,), jnp.float32),
        pltpu.SemaphoreType.DMA,
    ],
)
def _sc_score(heads, rels, tails, ent_re, ent_im, rel_re, rel_im,
              score_out, sq_out,
              idx_s, idx_vh, idx_vr, idx_vt, idx_sh,
              bhre, bhim, btre, btim, brre, brim,
              score_buf, sq_buf, sem):
    sid = lax.axis_index("s")
    wid = sid * _NC + lax.axis_index("c")
    base = wid * _BPW
    lane = lax.iota(jnp.int32, _L)

    sq_buf[...] = jnp.zeros((_L,), jnp.float32)

    pltpu.sync_copy(heads.at[pl.ds(base, _BPW)], idx_vh)
    pltpu.sync_copy(rels.at[pl.ds(base, _BPW)], idx_vr)
    pltpu.sync_copy(tails.at[pl.ds(base, _BPW)], idx_vt)
    pltpu.sync_copy(idx_vh, idx_sh.at[sid, pl.ds(0, _BPW)])
    pltpu.sync_copy(idx_vr, idx_sh.at[sid, pl.ds(_BPW, _BPW)])
    pltpu.sync_copy(idx_vt, idx_sh.at[sid, pl.ds(2 * _BPW, _BPW)])
    pltpu.sync_copy(idx_sh.at[sid], idx_s)

    def chunk_body(ci, _):
        off = base + ci * _C
        loc = ci * _C

        def issue_body(i, _):
            h = idx_s[loc + i]
            r = idx_s[_BPW + loc + i]
            t = idx_s[2 * _BPW + loc + i]
            pltpu.async_copy(ent_re.at[h], bhre.at[i], sem)
            pltpu.async_copy(ent_im.at[h], bhim.at[i], sem)
            pltpu.async_copy(ent_re.at[t], btre.at[i], sem)
            pltpu.async_copy(ent_im.at[t], btim.at[i], sem)
            pltpu.async_copy(rel_re.at[r], brre.at[i], sem)
            pltpu.async_copy(rel_im.at[r], brim.at[i], sem)
            return 0

        lax.fori_loop(0, _C, issue_body, 0)
        for buf in (bhre, bhim, btre, btim, brre, brim):
            pltpu.make_async_copy(ent_re.at[pl.ds(0, _C)], buf, sem).wait()

        def blk_body(blk, _):
            i0 = blk * _L
            rows = i0 + lane
            acc_re = jnp.zeros((_L,), jnp.float32)
            acc_im = jnp.zeros((_L,), jnp.float32)
            sq1 = jnp.zeros((_L,), jnp.float32)
            sq2 = jnp.zeros((_L,), jnp.float32)
            sq3 = jnp.zeros((_L,), jnp.float32)
            for d in range(_D):
                col = jnp.full((_L,), d, jnp.int32)
                hre = plsc.load_gather(bhre, [rows, col])
                him = plsc.load_gather(bhim, [rows, col])
                tre = plsc.load_gather(btre, [rows, col])
                tim = plsc.load_gather(btim, [rows, col])
                rre = plsc.load_gather(brre, [rows, col])
                rim = plsc.load_gather(brim, [rows, col])
                acc_re = acc_re + rre * (hre * tre + him * tim)
                acc_im = acc_im + rim * (hre * tim - him * tre)
                sq1 = sq1 + (hre * hre + him * him)
                sq2 = sq2 + (tre * tre + tim * tim)
                sq3 = sq3 + (rre * rre + rim * rim)
            score_buf[pl.ds(i0, _L)] = -(acc_re + acc_im)
            sq_buf[...] = sq_buf[...] + (sq1 + sq2 + sq3)
            return 0

        lax.fori_loop(0, _C // _L, blk_body, 0)
        pltpu.sync_copy(score_buf, score_out.at[pl.ds(off, _C)])
        return 0

    lax.fori_loop(0, _NCHUNK, chunk_body, 0)
    pltpu.sync_copy(sq_buf, sq_out.at[wid])


def _tc_finish_body(score_ref, labels_ref, sq_ref, out_ref):
    x = score_ref[...] * labels_ref[...]
    sp = jnp.maximum(x, 0.0) + jnp.log(1.0 + jnp.exp(-jnp.abs(x)))
    regul = jnp.sum(sq_ref[...]) * (0.01 / (_B * _D))
    total = jnp.sum(sp) * (1.0 / _B) + regul
    out_ref[...] = jnp.broadcast_to(total, (1, 1))


def _tc_finish(score2d, labels2d, sq):
    return pl.pallas_call(
        _tc_finish_body,
        out_shape=jax.ShapeDtypeStruct((1, 1), jnp.float32),
    )(score2d, labels2d, sq)


def kernel(batch, labels, ent_re, ent_im, rel_re, rel_im):
    heads = batch[:, 0]
    rels = batch[:, 1]
    tails = batch[:, 2]
    score, sq = _sc_score(heads, rels, tails, ent_re, ent_im, rel_re, rel_im)
    loss = _tc_finish(score.reshape(128, 128), labels.reshape(128, 128), sq)
    return loss[0, 0]


# speedup vs baseline: 1.8271x; 1.4432x over previous
import functools

import jax
import jax.numpy as jnp
from jax import lax
from jax.experimental import pallas as pl
from jax.experimental.pallas import tpu as pltpu
from jax.experimental.pallas import tpu_sc as plsc

_NUM_ENT = 100000
_NUM_REL = 100000
_D = 64
_B = 16384
_L = 16
_NC = 2
_NS = 16
_NW = _NC * _NS
_BPW = _B // _NW
_C = 128
_NCHUNK = _BPW // _C
_G = _D // _L

_mesh = plsc.VectorSubcoreMesh(core_axis_name="c", subcore_axis_name="s")


@functools.partial(
    pl.kernel,
    mesh=_mesh,
    compiler_params=pltpu.CompilerParams(needs_layout_passes=False),
    out_type=[
        jax.ShapeDtypeStruct((_B * _L,), jnp.float32),
        jax.ShapeDtypeStruct((_NW, _L), jnp.float32),
    ],
    scratch_types=[
        pltpu.SMEM((3 * _BPW,), jnp.int32),
        pltpu.VMEM((_BPW,), jnp.int32),
        pltpu.VMEM((_BPW,), jnp.int32),
        pltpu.VMEM((_BPW,), jnp.int32),
        pltpu.VMEM_SHARED((_NS, 3 * _BPW), jnp.int32),
        pltpu.VMEM((_C, _D), jnp.float32),
        pltpu.VMEM((_C, _D), jnp.float32),
        pltpu.VMEM((_C, _D), jnp.float32),
        pltpu.VMEM((_C, _D), jnp.float32),
        pltpu.VMEM((_C, _D), jnp.float32),
        pltpu.VMEM((_C, _D), jnp.float32),
        pltpu.VMEM((_C * _L,), jnp.float32),
        pltpu.VMEM((_L,), jnp.float32),
        pltpu.SemaphoreType.DMA,
    ],
)
def _sc_score(heads, rels, tails, ent_re, ent_im, rel_re, rel_im,
              score_out, sq_out,
              idx_s, idx_vh, idx_vr, idx_vt, idx_sh,
              bhre, bhim, btre, btim, brre, brim,
              score_buf, sq_buf, sem):
    sid = lax.axis_index("s")
    wid = sid * _NC + lax.axis_index("c")
    base = wid * _BPW

    pltpu.sync_copy(heads.at[pl.ds(base, _BPW)], idx_vh)
    pltpu.sync_copy(rels.at[pl.ds(base, _BPW)], idx_vr)
    pltpu.sync_copy(tails.at[pl.ds(base, _BPW)], idx_vt)
    pltpu.sync_copy(idx_vh, idx_sh.at[sid, pl.ds(0, _BPW)])
    pltpu.sync_copy(idx_vr, idx_sh.at[sid, pl.ds(_BPW, _BPW)])
    pltpu.sync_copy(idx_vt, idx_sh.at[sid, pl.ds(2 * _BPW, _BPW)])
    pltpu.sync_copy(idx_sh.at[sid], idx_s)

    def chunk_body(ci, sq_carry):
        off = base + ci * _C
        loc = ci * _C

        def issue_body(i, _):
            h = idx_s[loc + i]
            r = idx_s[_BPW + loc + i]
            t = idx_s[2 * _BPW + loc + i]
            pltpu.async_copy(ent_re.at[h], bhre.at[i], sem)
            pltpu.async_copy(ent_im.at[h], bhim.at[i], sem)
            pltpu.async_copy(ent_re.at[t], btre.at[i], sem)
            pltpu.async_copy(ent_im.at[t], btim.at[i], sem)
            pltpu.async_copy(rel_re.at[r], brre.at[i], sem)
            pltpu.async_copy(rel_im.at[r], brim.at[i], sem)
            return 0

        lax.fori_loop(0, _C, issue_body, 0)
        for buf in (bhre, bhim, btre, btim, brre, brim):
            pltpu.make_async_copy(ent_re.at[pl.ds(0, _C)], buf, sem).wait()

        def tri_body(i, carry):
            sq1, sq2, sq3 = carry
            score16 = jnp.zeros((_L,), jnp.float32)
            for g in range(_G):
                sl = pl.ds(g * _L, _L)
                hre = bhre[i, sl]
                him = bhim[i, sl]
                tre = btre[i, sl]
                tim = btim[i, sl]
                rre = brre[i, sl]
                rim = brim[i, sl]
                score16 = score16 + rre * (hre * tre + him * tim)
                score16 = score16 + rim * (hre * tim - him * tre)
                sq1 = sq1 + (hre * hre + him * him)
                sq2 = sq2 + (tre * tre + tim * tim)
                sq3 = sq3 + (rre * rre + rim * rim)
            score_buf[pl.ds(i * _L, _L)] = score16
            return (sq1, sq2, sq3)

        sq_carry = lax.fori_loop(0, _C, tri_body, sq_carry)
        pltpu.sync_copy(score_buf, score_out.at[pl.ds(off * _L, _C * _L)])
        return sq_carry

    zero = jnp.zeros((_L,), jnp.float32)
    sq1, sq2, sq3 = lax.fori_loop(0, _NCHUNK, chunk_body, (zero, zero, zero))
    sq_buf[...] = sq1 + sq2 + sq3
    pltpu.sync_copy(sq_buf, sq_out.at[wid])


def _tc_finish_body(part_ref, labels_ref, sq_ref, out_ref):
    part = part_ref[...]
    row = lax.broadcasted_iota(jnp.int32, (128, 8), 0)
    col = lax.broadcasted_iota(jnp.int32, (128, 8), 1)
    sel = (row // _L == col).astype(jnp.float32)
    score8 = -jax.lax.dot_general(
        part, sel, (((1,), (0,)), ((), ())),
        preferred_element_type=jnp.float32)
    x = score8 * labels_ref[...]
    sp = jnp.maximum(x, 0.0) + jnp.log(1.0 + jnp.exp(-jnp.abs(x)))
    regul = jnp.sum(sq_ref[...]) * (0.01 / (_B * _D))
    total = jnp.sum(sp) * (1.0 / _B) + regul
    out_ref[...] = jnp.broadcast_to(total, (1, 1))


def _tc_finish(part, labels8, sq):
    return pl.pallas_call(
        _tc_finish_body,
        out_shape=jax.ShapeDtypeStruct((1, 1), jnp.float32),
    )(part, labels8, sq)


def kernel(batch, labels, ent_re, ent_im, rel_re, rel_im):
    heads = batch[:, 0]
    rels = batch[:, 1]
    tails = batch[:, 2]
    part, sq = _sc_score(heads, rels, tails, ent_re, ent_im, rel_re, rel_im)
    loss = _tc_finish(part.reshape(_B // 8, 128), labels.reshape(_B // 8, 8), sq)
    return loss[0, 0]


# speedup vs baseline: 1.8334x; 1.0035x over previous
import functools

import jax
import jax.numpy as jnp
from jax import lax
from jax.experimental import pallas as pl
from jax.experimental.pallas import tpu as pltpu
from jax.experimental.pallas import tpu_sc as plsc

_NUM_ENT = 100000
_NUM_REL = 100000
_D = 64
_B = 16384
_L = 16
_NC = 2
_NS = 16
_NW = _NC * _NS
_BPW = _B // _NW
_C = 128
_NCHUNK = _BPW // _C
_G = _D // _L

_mesh = plsc.VectorSubcoreMesh(core_axis_name="c", subcore_axis_name="s")


@functools.partial(
    pl.kernel,
    mesh=_mesh,
    compiler_params=pltpu.CompilerParams(
        needs_layout_passes=False, use_tc_tiling_on_sc=True),
    out_type=[
        jax.ShapeDtypeStruct((_B * _L,), jnp.float32),
        jax.ShapeDtypeStruct((_NW, _L), jnp.float32),
    ],
    scratch_types=[
        pltpu.SMEM((3 * _BPW,), jnp.int32),
        pltpu.VMEM((_BPW,), jnp.int32),
        pltpu.VMEM((_BPW,), jnp.int32),
        pltpu.VMEM((_BPW,), jnp.int32),
        pltpu.VMEM_SHARED((_NS, 3 * _BPW), jnp.int32),
        pltpu.VMEM((_C, _D), jnp.float32),
        pltpu.VMEM((_C, _D), jnp.float32),
        pltpu.VMEM((_C, _D), jnp.float32),
        pltpu.VMEM((_C, _D), jnp.float32),
        pltpu.VMEM((_C, _D), jnp.float32),
        pltpu.VMEM((_C, _D), jnp.float32),
        pltpu.VMEM((_C * _L,), jnp.float32),
        pltpu.VMEM((_L,), jnp.float32),
        pltpu.SemaphoreType.DMA,
    ],
)
def _sc_score(heads, rels, tails, ent_re, ent_im, rel_re, rel_im,
              score_out, sq_out,
              idx_s, idx_vh, idx_vr, idx_vt, idx_sh,
              bhre, bhim, btre, btim, brre, brim,
              score_buf, sq_buf, sem):
    sid = lax.axis_index("s")
    wid = sid * _NC + lax.axis_index("c")
    base = wid * _BPW

    pltpu.sync_copy(heads.at[pl.ds(base, _BPW)], idx_vh)
    pltpu.sync_copy(rels.at[pl.ds(base, _BPW)], idx_vr)
    pltpu.sync_copy(tails.at[pl.ds(base, _BPW)], idx_vt)
    pltpu.sync_copy(idx_vh, idx_sh.at[sid, pl.ds(0, _BPW)])
    pltpu.sync_copy(idx_vr, idx_sh.at[sid, pl.ds(_BPW, _BPW)])
    pltpu.sync_copy(idx_vt, idx_sh.at[sid, pl.ds(2 * _BPW, _BPW)])
    pltpu.sync_copy(idx_sh.at[sid], idx_s)

    def chunk_body(ci, sq_carry):
        off = base + ci * _C
        loc = ci * _C

        def issue_body(i, _):
            h = idx_s[loc + i]
            r = idx_s[_BPW + loc + i]
            t = idx_s[2 * _BPW + loc + i]
            pltpu.async_copy(ent_re.at[h], bhre.at[i], sem)
            pltpu.async_copy(ent_im.at[h], bhim.at[i], sem)
            pltpu.async_copy(ent_re.at[t], btre.at[i], sem)
            pltpu.async_copy(ent_im.at[t], btim.at[i], sem)
            pltpu.async_copy(rel_re.at[r], brre.at[i], sem)
            pltpu.async_copy(rel_im.at[r], brim.at[i], sem)
            return 0

        lax.fori_loop(0, _C, issue_body, 0)
        for buf in (bhre, bhim, btre, btim, brre, brim):
            pltpu.make_async_copy(ent_re.at[pl.ds(0, _C)], buf, sem).wait()

        def tri_body(i, carry):
            sq1, sq2, sq3 = carry
            score16 = jnp.zeros((_L,), jnp.float32)
            for g in range(_G):
                sl = pl.ds(g * _L, _L)
                hre = bhre[i, sl]
                him = bhim[i, sl]
                tre = btre[i, sl]
                tim = btim[i, sl]
                rre = brre[i, sl]
                rim = brim[i, sl]
                score16 = score16 + rre * (hre * tre + him * tim)
                score16 = score16 + rim * (hre * tim - him * tre)
                sq1 = sq1 + (hre * hre + him * him)
                sq2 = sq2 + (tre * tre + tim * tim)
                sq3 = sq3 + (rre * rre + rim * rim)
            score_buf[pl.ds(i * _L, _L)] = score16
            return (sq1, sq2, sq3)

        sq_carry = lax.fori_loop(0, _C, tri_body, sq_carry)
        pltpu.sync_copy(score_buf, score_out.at[pl.ds(off * _L, _C * _L)])
        return sq_carry

    zero = jnp.zeros((_L,), jnp.float32)
    sq1, sq2, sq3 = lax.fori_loop(0, _NCHUNK, chunk_body, (zero, zero, zero))
    sq_buf[...] = sq1 + sq2 + sq3
    pltpu.sync_copy(sq_buf, sq_out.at[wid])


def _tc_finish_body(part_ref, labels_ref, sq_ref, out_ref):
    part = part_ref[...]
    row = lax.broadcasted_iota(jnp.int32, (128, 8), 0)
    col = lax.broadcasted_iota(jnp.int32, (128, 8), 1)
    sel = (row // _L == col).astype(jnp.float32)
    score8 = -jax.lax.dot_general(
        part, sel, (((1,), (0,)), ((), ())),
        preferred_element_type=jnp.float32)
    x = score8 * labels_ref[...]
    sp = jnp.maximum(x, 0.0) + jnp.log(1.0 + jnp.exp(-jnp.abs(x)))
    regul = jnp.sum(sq_ref[...]) * (0.01 / (_B * _D))
    total = jnp.sum(sp) * (1.0 / _B) + regul
    out_ref[...] = jnp.broadcast_to(total, (1, 1))


def _tc_finish(part, labels8, sq):
    return pl.pallas_call(
        _tc_finish_body,
        out_shape=jax.ShapeDtypeStruct((1, 1), jnp.float32),
    )(part, labels8, sq)


def kernel(batch, labels, ent_re, ent_im, rel_re, rel_im):
    heads = batch[:, 0]
    rels = batch[:, 1]
    tails = batch[:, 2]
    part, sq = _sc_score(heads, rels, tails, ent_re, ent_im, rel_re, rel_im)
    loss = _tc_finish(part.reshape(_B // 8, 128), labels.reshape(_B // 8, 8), sq)
    return loss[0, 0]
